# baseline (speedup 1.0000x reference)
import jax
import jax.numpy as jnp
from jax.experimental import pallas as pl


def _vq_body(x_ref, e_ref, x2_ref, e2_ref, o_ref):
    xn = x_ref[...]
    e = e_ref[...]
    dots = jax.lax.dot_general(
        xn.astype(jnp.bfloat16), e.astype(jnp.bfloat16),
        (((1,), (1,)), ((), ())),
        preferred_element_type=jnp.float32)
    dist = -(x2_ref[...] - 2.0 * dots + e2_ref[...])
    half = dist.shape[1] // 2
    h1 = dist[:, :half]
    h2 = dist[:, half:]
    m1 = jnp.max(h1, axis=1)
    i1 = jnp.argmax(h1, axis=1).astype(jnp.int32)
    m2 = jnp.max(h2, axis=1)
    i2 = jnp.argmax(h2, axis=1).astype(jnp.int32) + half
    m1r = m1.astype(jnp.bfloat16).astype(jnp.float32)
    o_ref[...] = jnp.where(m2 > m1r, i2, i1)


def kernel(x, embed):
    B, T, D = x.shape
    K = embed.shape[0]
    N = B * T
    norm = jnp.linalg.norm(x, ord=2, axis=-1, keepdims=True)
    xn = x / jnp.maximum(norm, 1e-12)
    flat = xn.reshape(N, D)
    embed_t = embed.T
    x2 = jnp.sum(flat ** 2, axis=1, keepdims=True)
    e2 = jnp.sum(embed_t ** 2, axis=0, keepdims=True)
    BT = 256
    out = pl.pallas_call(
        _vq_body,
        grid=(N // BT,),
        in_specs=[
            pl.BlockSpec((BT, D), lambda i: (i, 0)),
            pl.BlockSpec((K, D), lambda i: (0, 0)),
            pl.BlockSpec((BT, 1), lambda i: (i, 0)),
            pl.BlockSpec((1, K), lambda i: (0, 0)),
        ],
        out_specs=pl.BlockSpec((BT,), lambda i: (i,)),
        out_shape=jax.ShapeDtypeStruct((N,), jnp.int32),
    )(flat, embed, x2, e2)
    return out.reshape(B, T)


# speedup vs baseline: 1.1749x; 1.1749x over previous
import jax
import jax.numpy as jnp
from jax.experimental import pallas as pl


def _vq_body(x_ref, e_ref, x2_ref, e2_ref, o_ref):
    xn = x_ref[...]
    e = e_ref[...]
    dots2 = jax.lax.dot_general(
        (xn * 2.0).astype(jnp.bfloat16), e.astype(jnp.bfloat16),
        (((1,), (1,)), ((), ())),
        preferred_element_type=jnp.float32)
    dist = -(x2_ref[...] - dots2 + e2_ref[...])
    half = dist.shape[1] // 2
    h1 = dist[:, :half]
    h2 = dist[:, half:]
    m1 = jnp.max(h1, axis=1)
    i1 = jnp.argmax(h1, axis=1).astype(jnp.int32)
    m2 = jnp.max(h2, axis=1)
    i2 = jnp.argmax(h2, axis=1).astype(jnp.int32) + half
    m1r = m1.astype(jnp.bfloat16).astype(jnp.float32)
    o_ref[...] = jnp.where(m2 > m1r, i2, i1)


def kernel(x, embed):
    B, T, D = x.shape
    K = embed.shape[0]
    N = B * T
    norm = jnp.linalg.norm(x, ord=2, axis=-1, keepdims=True)
    xn = x / jnp.maximum(norm, 1e-12)
    flat = xn.reshape(N, D)
    embed_t = embed.T
    x2 = jnp.sum(flat ** 2, axis=1, keepdims=True)
    e2 = jnp.sum(embed_t ** 2, axis=0, keepdims=True)
    BT = 512
    out = pl.pallas_call(
        _vq_body,
        grid=(N // BT,),
        in_specs=[
            pl.BlockSpec((BT, D), lambda i: (i, 0)),
            pl.BlockSpec((K, D), lambda i: (0, 0)),
            pl.BlockSpec((BT, 1), lambda i: (i, 0)),
            pl.BlockSpec((1, K), lambda i: (0, 0)),
        ],
        out_specs=pl.BlockSpec((BT,), lambda i: (i,)),
        out_shape=jax.ShapeDtypeStruct((N,), jnp.int32),
    )(flat, embed, x2, e2)
    return out.reshape(B, T)
